# Initial kernel scaffold; baseline (speedup 1.0000x reference)
import jax
import jax.numpy as jnp
from jax.experimental import pallas as pl


def kernel(batch_garment_verts, batch_body_verts, body_faces):
    raise NotImplementedError("write your pallas kernel here")


# speedup vs baseline: 7.5832x; 7.5832x over previous
import functools

import jax
import jax.numpy as jnp
import numpy as np
from jax.experimental import pallas as pl
from jax.experimental.pallas import tpu as pltpu

P_BLK = 256
T_BLK = 32


def _nearest_body(pts_ref, tri_ref, out_ref, *, t_pad, f):
    px = pts_ref[0, 0:1, :]
    py = pts_ref[0, 1:2, :]
    pz = pts_ref[0, 2:3, :]

    t_iota = jax.lax.broadcasted_iota(jnp.int32, (T_BLK, P_BLK), 0)
    eps = 1e-12

    def body(t, carry):
        best_sq, best_idx = carry
        base = t * T_BLK

        def s(k):
            return tri_ref[0, pl.ds(base, T_BLK), k:k + 1]

        abx, aby, abz = s(0), s(1), s(2)
        acx, acy, acz = s(3), s(4), s(5)
        ax, ay, az = s(6), s(7), s(8)
        abab, abac, acac = s(9), s(10), s(11)
        inv_abab, inv_acac, inv_bcbc = s(12), s(13), s(14)

        apx = px - ax
        apy = py - ay
        apz = pz - az
        d1 = abx * apx + aby * apy + abz * apz
        d2 = acx * apx + acy * apy + acz * apz
        d3 = d1 - abab
        d4 = d2 - abac
        d5 = d1 - abac
        d6 = d2 - acac
        vc = d1 * d4 - d3 * d2
        vb = d5 * d2 - d1 * d6
        va = d3 * d6 - d5 * d4

        cond_a = (d1 <= 0.0) & (d2 <= 0.0)
        cond_b = (d3 >= 0.0) & (d4 <= d3)
        cond_c = (d6 >= 0.0) & (d5 <= d6)
        cond_ab = (vc <= 0.0) & (d1 >= 0.0) & (d3 <= 0.0)
        cond_ac = (vb <= 0.0) & (d2 >= 0.0) & (d6 <= 0.0)
        u1 = d4 - d3
        u2 = d5 - d6
        cond_bc = (va <= 0.0) & (u1 >= 0.0) & (u2 >= 0.0)

        t_ab = d1 * inv_abab
        t_ac = d2 * inv_acac
        t_bc = u1 * inv_bcbc
        r = 1.0 / jnp.maximum(va + vb + vc, eps)
        zero = jnp.zeros_like(d1)
        one = jnp.ones_like(d1)
        sb = vb * r
        wb = vc * r
        sb = jnp.where(cond_bc, 1.0 - t_bc, sb)
        wb = jnp.where(cond_bc, t_bc, wb)
        sb = jnp.where(cond_ac, zero, sb)
        wb = jnp.where(cond_ac, t_ac, wb)
        sb = jnp.where(cond_ab, t_ab, sb)
        wb = jnp.where(cond_ab, zero, wb)
        sb = jnp.where(cond_c, zero, sb)
        wb = jnp.where(cond_c, one, wb)
        sb = jnp.where(cond_b, one, sb)
        wb = jnp.where(cond_b, zero, wb)
        sb = jnp.where(cond_a, zero, sb)
        wb = jnp.where(cond_a, zero, wb)

        dx = apx - sb * abx - wb * acx
        dy = apy - sb * aby - wb * acy
        dz = apz - sb * abz - wb * acz
        sq = dx * dx + dy * dy + dz * dz

        idx = t_iota + base
        lt = sq < best_sq
        return jnp.where(lt, sq, best_sq), jnp.where(lt, idx, best_idx)

    init = (jnp.full((T_BLK, P_BLK), jnp.inf, jnp.float32),
            jnp.full((T_BLK, P_BLK), jnp.int32(2 ** 30), jnp.int32))
    best_sq, best_idx = jax.lax.fori_loop(0, t_pad // T_BLK, body, init)

    m = jnp.min(best_sq, axis=0, keepdims=True)
    cand = jnp.where(best_sq == m, best_idx, jnp.int32(2 ** 30))
    out_ref[0, 0] = jnp.min(cand, axis=0, keepdims=True)


def _pack_tris(bv, faces, t_pad):
    a = bv[faces[:, 0]]
    b = bv[faces[:, 1]]
    c = bv[faces[:, 2]]
    ab = b - a
    ac = c - a
    abab = jnp.sum(ab * ab, -1)
    abac = jnp.sum(ab * ac, -1)
    acac = jnp.sum(ac * ac, -1)
    bcbc = abab - 2.0 * abac + acac
    eps = 1e-12
    cols = [
        ab[:, 0], ab[:, 1], ab[:, 2],
        ac[:, 0], ac[:, 1], ac[:, 2],
        a[:, 0], a[:, 1], a[:, 2],
        abab, abac, acac,
        1.0 / jnp.maximum(abab, eps), 1.0 / jnp.maximum(acac, eps),
        1.0 / jnp.maximum(bcbc, eps),
        jnp.zeros_like(abab),
    ]
    tri = jnp.stack(cols, axis=-1)
    f = faces.shape[0]
    pad = jnp.zeros((t_pad - f, 16), jnp.float32)
    pad = pad.at[:, 6:9].set(1e4)
    return jnp.concatenate([tri, pad], axis=0)


def _nearest_pallas(g_pts, tri_packed, t_pad):
    n = g_pts.shape[2]
    npb = n // P_BLK
    f = 0
    grid = (2, npb)
    out = pl.pallas_call(
        functools.partial(_nearest_body, t_pad=t_pad, f=f),
        grid=grid,
        in_specs=[
            pl.BlockSpec((1, 3, P_BLK), lambda b, p: (b, 0, p)),
            pl.BlockSpec((1, t_pad, 16), lambda b, p: (b, 0, 0)),
        ],
        out_specs=pl.BlockSpec((1, 1, 1, P_BLK), lambda b, p: (b, p, 0, 0)),
        out_shape=jax.ShapeDtypeStruct((2, npb, 1, P_BLK), jnp.int32),
    )(g_pts, tri_packed)
    return out.reshape(2, n)


def _closest_point_tri_ref(p, a, b, c):
    ab = b - a
    ac = c - a
    ap = p - a
    d1 = jnp.sum(ab * ap, -1)
    d2 = jnp.sum(ac * ap, -1)
    bp = p - b
    d3 = jnp.sum(ab * bp, -1)
    d4 = jnp.sum(ac * bp, -1)
    cp = p - c
    d5 = jnp.sum(ab * cp, -1)
    d6 = jnp.sum(ac * cp, -1)
    vc = d1 * d4 - d3 * d2
    vb = d5 * d2 - d1 * d6
    va = d3 * d6 - d5 * d4
    eps = 1e-12
    cond_a = (d1 <= 0) & (d2 <= 0)
    cond_b = (d3 >= 0) & (d4 <= d3)
    cond_c = (d6 >= 0) & (d5 <= d6)
    cond_ab = (vc <= 0) & (d1 >= 0) & (d3 <= 0)
    cond_ac = (vb <= 0) & (d2 >= 0) & (d6 <= 0)
    cond_bc = (va <= 0) & ((d4 - d3) >= 0) & ((d5 - d6) >= 0)
    t_ab = d1 / jnp.maximum(d1 - d3, eps)
    pt_ab = a + t_ab[..., None] * ab
    t_ac = d2 / jnp.maximum(d2 - d6, eps)
    pt_ac = a + t_ac[..., None] * ac
    t_bc = (d4 - d3) / jnp.maximum((d4 - d3) + (d5 - d6), eps)
    pt_bc = b + t_bc[..., None] * (c - b)
    denom = jnp.maximum(va + vb + vc, eps)
    v = (vb / denom)[..., None]
    w = (vc / denom)[..., None]
    pt = a + v * ab + w * ac
    part = jnp.zeros(pt.shape[:-1], dtype=jnp.int32)
    pt = jnp.where(cond_bc[..., None], pt_bc, pt)
    part = jnp.where(cond_bc, 2, part)
    pt = jnp.where(cond_ac[..., None], pt_ac, pt)
    part = jnp.where(cond_ac, 3, part)
    pt = jnp.where(cond_ab[..., None], pt_ab, pt)
    part = jnp.where(cond_ab, 1, part)
    pt = jnp.where(cond_c[..., None], c, pt)
    part = jnp.where(cond_c, 6, part)
    pt = jnp.where(cond_b[..., None], b, pt)
    part = jnp.where(cond_b, 5, part)
    pt = jnp.where(cond_a[..., None], a, pt)
    part = jnp.where(cond_a, 4, part)
    return pt, part


def _finalize(g_verts, b_verts, faces, ntri):
    a = b_verts[faces[:, 0]]
    b = b_verts[faces[:, 1]]
    c = b_verts[faces[:, 2]]
    fn_un = jnp.cross(b - a, c - a)
    fn = fn_un / (jnp.linalg.norm(fn_un, axis=-1, keepdims=True) + 1e-10)
    vn = jnp.zeros_like(b_verts)
    vn = vn.at[faces[:, 0]].add(fn).at[faces[:, 1]].add(fn).at[faces[:, 2]].add(fn)
    vn = vn / (jnp.linalg.norm(vn, axis=-1, keepdims=True) + 1e-10)

    ta = a[ntri]
    tb = b[ntri]
    tc = c[ntri]
    npt, npart = _closest_point_tri_ref(g_verts, ta, tb, tc)

    tri_mask = npart == 0
    vert_mask = npart > 3
    edge_mask = (npart >= 1) & (npart <= 3)
    nn = jnp.zeros_like(npt)
    nn = jnp.where(tri_mask[:, None], fn[ntri], nn)
    vidx = jnp.clip(npart - 4, 0, 2)
    nn = jnp.where(vert_mask[:, None], vn[faces[ntri, vidx]], nn)
    e1 = jnp.clip(npart - 1, 0, 2)
    e2 = jnp.mod(npart, 3)
    esum = vn[faces[ntri, e1]] + vn[faces[ntri, e2]]
    nn = jnp.where(edge_mask[:, None], esum, nn)
    nn = nn / (jnp.linalg.norm(nn, axis=-1, keepdims=True) + 1e-10)
    dist = jax.nn.relu(jnp.sum(-(g_verts - npt) * nn, axis=1))
    return dist.sum()


def kernel(batch_garment_verts, batch_body_verts, body_faces):
    faces = body_faces.astype(jnp.int32)
    f = faces.shape[0]
    t_pad = ((f + T_BLK - 1) // T_BLK) * T_BLK

    g = batch_garment_verts
    g_pts = jnp.swapaxes(g, 1, 2)

    tri_packed = jnp.stack(
        [_pack_tris(batch_body_verts[i], faces, t_pad) for i in range(2)], axis=0)

    ntri = _nearest_pallas(g_pts, tri_packed, t_pad)

    losses = [
        _finalize(batch_garment_verts[i], batch_body_verts[i], faces, ntri[i])
        for i in range(2)
    ]
    return jnp.stack(losses)
